# baseline (speedup 1.0000x reference)
import jax
import jax.numpy as jnp
from jax import lax
from jax.experimental import pallas as pl
from jax.experimental.pallas import tpu as pltpu

_FEAT = 30
_RB = 8192


def _round_up(x, m):
    return ((x + m - 1) // m) * m


def _affine_tanh_kernel(x_ref, w_ref, c_ref, o_ref):
    y = lax.dot_general(
        x_ref[...], w_ref[...],
        dimension_numbers=(((1,), (0,)), ((), ())),
        preferred_element_type=jnp.float32,
    )
    o_ref[...] = jnp.tanh(y + c_ref[0])


def kernel(features, w1, b1, w2, b2, w3, b3):
    B = features.shape[0]
    x = features.astype(jnp.float32)

    w_col = (w3 @ w2 @ w1).reshape(_FEAT, 1).astype(jnp.float32)
    c = (b1 @ w2.T @ w3.T + b2 @ w3.T + b3).reshape(1).astype(jnp.float32)

    B_pad = _round_up(B, 8)
    if B_pad != B:
        x = jnp.pad(x, ((0, B_pad - B), (0, 0)))

    if B_pad > _RB:
        rb = _RB
    elif B_pad >= 16:
        rb = _round_up(pl.cdiv(B_pad, 2), 8)
    else:
        rb = B_pad
    num_tiles = pl.cdiv(B_pad, rb)

    out = pl.pallas_call(
        _affine_tanh_kernel,
        out_shape=jax.ShapeDtypeStruct((B_pad, 1), jnp.float32),
        grid=(num_tiles,),
        in_specs=[
            pl.BlockSpec((rb, _FEAT), lambda i: (i, 0)),
            pl.BlockSpec((_FEAT, 1), lambda i: (0, 0)),
            pl.BlockSpec(memory_space=pltpu.MemorySpace.SMEM),
        ],
        out_specs=pl.BlockSpec((rb, 1), lambda i: (i, 0)),
        compiler_params=pltpu.CompilerParams(
            dimension_semantics=("parallel",),
        ),
    )(x, w_col, c)

    return out if B_pad == B else out[:B]


# speedup vs baseline: 11.1920x; 11.1920x over previous
import jax
import jax.numpy as jnp
from jax.experimental import pallas as pl
from jax.experimental.pallas import tpu as pltpu

_FEAT = 30
_LB = 32768


def _affine_tanh_kernel(x_ref, w_ref, c_ref, o_ref):
    y = jnp.sum(x_ref[...] * w_ref[...], axis=0, keepdims=True)
    o_ref[...] = jnp.tanh(y + c_ref[0])


def kernel(features, w1, b1, w2, b2, w3, b3):
    B = features.shape[0]
    x_t = features.astype(jnp.float32).T

    w_col = (w3 @ w2 @ w1).reshape(_FEAT, 1).astype(jnp.float32)
    c = (b1 @ w2.T @ w3.T + b2 @ w3.T + b3).reshape(1).astype(jnp.float32)

    if B > _LB:
        lb = _LB
    elif B >= 256:
        lb = ((B // 2 + 127) // 128) * 128
    else:
        lb = B
    num_tiles = pl.cdiv(B, lb)

    out = pl.pallas_call(
        _affine_tanh_kernel,
        out_shape=jax.ShapeDtypeStruct((1, B), jnp.float32),
        grid=(num_tiles,),
        in_specs=[
            pl.BlockSpec((_FEAT, lb), lambda i: (0, i)),
            pl.BlockSpec((_FEAT, 1), lambda i: (0, 0)),
            pl.BlockSpec(memory_space=pltpu.MemorySpace.SMEM),
        ],
        out_specs=pl.BlockSpec((1, lb), lambda i: (0, i)),
        compiler_params=pltpu.CompilerParams(
            dimension_semantics=("parallel",),
        ),
    )(x_t, w_col, c)

    return out.reshape(B, 1)


# speedup vs baseline: 11.7323x; 1.0483x over previous
import jax
import jax.numpy as jnp
from jax.experimental import pallas as pl
from jax.experimental.pallas import tpu as pltpu

_FEAT = 30
_LB = 65536


def _affine_tanh_kernel(x_ref, w_ref, c_ref, o_ref):
    y = jnp.sum(x_ref[...] * w_ref[...], axis=0, keepdims=True)
    o_ref[...] = jnp.tanh(y + c_ref[0])


def kernel(features, w1, b1, w2, b2, w3, b3):
    B = features.shape[0]
    x_t = features.astype(jnp.float32).T

    w_col = (w3 @ w2 @ w1).reshape(_FEAT, 1).astype(jnp.float32)
    c = (b1 @ w2.T @ w3.T + b2 @ w3.T + b3).reshape(1).astype(jnp.float32)

    if B > _LB:
        lb = _LB
    elif B >= 256:
        lb = ((B // 2 + 127) // 128) * 128
    else:
        lb = B
    num_tiles = pl.cdiv(B, lb)

    out = pl.pallas_call(
        _affine_tanh_kernel,
        out_shape=jax.ShapeDtypeStruct((1, B), jnp.float32),
        grid=(num_tiles,),
        in_specs=[
            pl.BlockSpec((_FEAT, lb), lambda i: (0, i)),
            pl.BlockSpec((_FEAT, 1), lambda i: (0, 0)),
            pl.BlockSpec(memory_space=pltpu.MemorySpace.SMEM),
        ],
        out_specs=pl.BlockSpec((1, lb), lambda i: (0, i)),
        compiler_params=pltpu.CompilerParams(
            dimension_semantics=("parallel",),
        ),
    )(x_t, w_col, c)

    return out.reshape(B, 1)
